# baseline (speedup 1.0000x reference)
import functools

import jax
import jax.numpy as jnp
from jax import lax
from jax.experimental import pallas as pl
from jax.experimental.pallas import tpu as pltpu

LANE = 128
TILE = 512
GROUPS = 16
UNROLL_J = 4


def _masked_log_softmax(logits, num_classes):
    col = lax.broadcasted_iota(jnp.int32, logits.shape, 1)
    logits = jnp.where(col < num_classes, logits, -1e30)
    mx = jnp.max(logits, axis=1, keepdims=True)
    z = logits - mx
    lse = jnp.log(jnp.sum(jnp.exp(z), axis=1, keepdims=True))
    return z - lse


def _build_kernel(bounds_ref, edges_ref, x_ref, w_ref,
                  adj_ref, d_ref, m0_ref, *scratch,
                  n, tile, e_pad):
    i = pl.program_id(0)
    wq = min(8 * LANE, n)
    nq = n // wq
    shift_q = (wq - 1).bit_length()
    rpg = tile // GROUPS
    grp_refs = scratch[:GROUPS]
    scr2_ref = scratch[GROUPS]
    dummy = rpg * nq

    for g in range(GROUPS):
        grp_refs[g][...] = jnp.zeros_like(grp_refs[g])

    base = i * GROUPS
    starts = [bounds_ref[base + g] for g in range(GROUPS)]
    ends = [bounds_ref[base + g + 1] for g in range(GROUPS)]
    trip = functools.reduce(jnp.maximum,
                            [ends[g] - starts[g] for g in range(GROUPS)])

    iota = lax.broadcasted_iota(jnp.int32, (1, wq), 1)

    def body(j, carry):
        for jj in range(UNROLL_J):
            vals, tgts, ohs = [], [], []
            for g in range(GROUPS):
                idx = starts[g] + j * UNROLL_J + jj
                pred = idx < ends[g]
                l = edges_ref[jnp.minimum(idx, e_pad - 1)]
                rq = (lax.shift_right_logical(l, shift_q)
                      - (i * tile + g * rpg) * nq)
                cq = lax.bitwise_and(l, wq - 1)
                tgt = jnp.where(pred, rq, dummy)
                v = grp_refs[g][tgt]
                oh = jnp.where(iota == cq, 1.0, 0.0)
                vals.append(v); tgts.append(tgt); ohs.append(oh)
            for g in range(GROUPS):
                grp_refs[g][tgts[g]] = vals[g] + ohs[g]
        return carry

    lax.fori_loop(0, (trip + UNROLL_J - 1) // UNROLL_J, body, 0)

    for g in range(GROUPS):
        scr2_ref[g * rpg:(g + 1) * rpg, :] = (
            grp_refs[g][:rpg * nq].reshape(rpg, n))
    slab = scr2_ref[...]
    adj_ref[...] = slab.astype(jnp.int8)
    deg = jnp.sum(slab, axis=1, keepdims=True) + 1.0
    d = lax.rsqrt(deg)
    d_ref[...] = d

    xb = x_ref[...].astype(jnp.bfloat16)
    xw = jnp.dot(xb, w_ref[...], preferred_element_type=jnp.float32)
    m0_ref[...] = (xw * d).astype(m0_ref.dtype)


def _prop_kernel(adj_ref, d_ref, m_ref, o_ref, *, tile):
    i = pl.program_id(0)
    a = adj_ref[...].astype(jnp.bfloat16)
    h = jnp.dot(a, m_ref[...], preferred_element_type=jnp.float32)
    start = pl.multiple_of(i * tile, tile)
    h += m_ref[pl.ds(start, tile), :].astype(jnp.float32)
    d = d_ref[...]
    o_ref[...] = (h * (d * d)).astype(o_ref.dtype)


def _prop_final_kernel(adj_ref, d_ref, m_ref, b_ref, o_ref, *, tile,
                       num_classes):
    i = pl.program_id(0)
    a = adj_ref[...].astype(jnp.bfloat16)
    h = jnp.dot(a, m_ref[...], preferred_element_type=jnp.float32)
    start = pl.multiple_of(i * tile, tile)
    h += m_ref[pl.ds(start, tile), :].astype(jnp.float32)
    logits = h * d_ref[...] + b_ref[...]
    o_ref[...] = _masked_log_softmax(logits, num_classes).astype(o_ref.dtype)


def kernel(x, edge_index, weight, bias):
    n, f = x.shape
    c = weight.shape[1]
    row, col = edge_index[0], edge_index[1]
    e = int(row.shape[0])
    t = min(TILE, n)
    cp = ((c + LANE - 1) // LANE) * LANE
    e_pad = max(1 << (e + 1 - 1).bit_length(), 1024)

    lin = row * n + col
    lin = jnp.concatenate(
        [lin, jnp.full((e_pad - e,), n * n, jnp.int32)])
    sl = lax.sort(lin, is_stable=False)
    rows_per_group = t // GROUPS
    n_groups = (n // t) * GROUPS
    boundaries = jnp.arange(n_groups + 1, dtype=jnp.int32) * (rows_per_group * n)
    bounds = jnp.sum(
        (lin[:, None] < boundaries[None, :]).astype(jnp.int32), axis=0,
        dtype=jnp.int32)

    w_p = jnp.zeros((f, cp), jnp.bfloat16).at[:, :c].set(
        weight.astype(jnp.bfloat16))
    b_p = jnp.zeros((1, cp), jnp.float32).at[0, :c].set(bias)

    grid_rows = n // t
    vmem_limit = 100 * 1024 * 1024

    adj, d_p, m0 = pl.pallas_call(
        functools.partial(_build_kernel, n=n, tile=t, e_pad=e_pad),
        out_shape=[jax.ShapeDtypeStruct((n, n), jnp.int8),
                   jax.ShapeDtypeStruct((n, 1), jnp.float32),
                   jax.ShapeDtypeStruct((n, cp), jnp.bfloat16)],
        grid_spec=pltpu.PrefetchScalarGridSpec(
            num_scalar_prefetch=2,
            grid=(grid_rows,),
            in_specs=[pl.BlockSpec((t, f), lambda i, *_: (i, 0)),
                      pl.BlockSpec((f, cp), lambda i, *_: (0, 0))],
            out_specs=[pl.BlockSpec((t, n), lambda i, *_: (i, 0)),
                       pl.BlockSpec((t, 1), lambda i, *_: (i, 0)),
                       pl.BlockSpec((t, cp), lambda i, *_: (i, 0))],
            scratch_shapes=(
                [pltpu.VMEM(
                    ((t // GROUPS) * (n // min(8 * LANE, n)) + 8, 1,
                     min(8 * LANE, n)), jnp.float32)
                 for _ in range(GROUPS)]
                + [pltpu.VMEM((t, n), jnp.float32)]),
        ),
        compiler_params=pltpu.CompilerParams(
            dimension_semantics=("parallel",),
            vmem_limit_bytes=vmem_limit),
        cost_estimate=pl.CostEstimate(
            flops=2 * n * f * cp + 8 * e, transcendentals=n,
            bytes_accessed=n * n + n * f * 4 + f * cp * 2 + n * cp * 2),
    )(bounds, sl, x, w_p)

    m1 = pl.pallas_call(
        functools.partial(_prop_kernel, tile=t),
        out_shape=jax.ShapeDtypeStruct((n, cp), jnp.bfloat16),
        grid=(grid_rows,),
        in_specs=[pl.BlockSpec((t, n), lambda i: (i, 0)),
                  pl.BlockSpec((t, 1), lambda i: (i, 0)),
                  pl.BlockSpec((n, cp), lambda i: (0, 0))],
        out_specs=pl.BlockSpec((t, cp), lambda i: (i, 0)),
        compiler_params=pltpu.CompilerParams(
            dimension_semantics=("parallel",),
            vmem_limit_bytes=vmem_limit),
        cost_estimate=pl.CostEstimate(
            flops=2 * n * n * cp, transcendentals=0,
            bytes_accessed=n * n + n * cp * 2 + n * 4 + n * cp * 2),
    )(adj, d_p, m0)

    out_p = pl.pallas_call(
        functools.partial(_prop_final_kernel, tile=t, num_classes=c),
        out_shape=jax.ShapeDtypeStruct((n, cp), jnp.float32),
        grid=(grid_rows,),
        in_specs=[pl.BlockSpec((t, n), lambda i: (i, 0)),
                  pl.BlockSpec((t, 1), lambda i: (i, 0)),
                  pl.BlockSpec((n, cp), lambda i: (0, 0)),
                  pl.BlockSpec((1, cp), lambda i: (0, 0))],
        out_specs=pl.BlockSpec((t, cp), lambda i: (i, 0)),
        compiler_params=pltpu.CompilerParams(
            dimension_semantics=("parallel",),
            vmem_limit_bytes=vmem_limit),
        cost_estimate=pl.CostEstimate(
            flops=2 * n * n * cp, transcendentals=n * cp,
            bytes_accessed=n * n + n * cp * 2 + n * 4 + cp * 4 + n * cp * 4),
    )(adj, d_p, m1, b_p)

    return out_p[:, :c]


# speedup vs baseline: 1.0728x; 1.0728x over previous
import functools

import jax
import jax.numpy as jnp
from jax import lax
from jax.experimental import pallas as pl
from jax.experimental.pallas import tpu as pltpu

LANE = 128
TILE = 512
GROUPS = 8
UNROLL_J = 16


def _masked_log_softmax(logits, num_classes):
    col = lax.broadcasted_iota(jnp.int32, logits.shape, 1)
    logits = jnp.where(col < num_classes, logits, -1e30)
    mx = jnp.max(logits, axis=1, keepdims=True)
    z = logits - mx
    lse = jnp.log(jnp.sum(jnp.exp(z), axis=1, keepdims=True))
    return z - lse


def _build_kernel(bounds_ref, edges_ref, x_ref, w_ref,
                  adj_ref, d_ref, m0_ref, *scratch,
                  n, tile, e_pad):
    i = pl.program_id(0)
    wq = min(8 * LANE, n)
    nq = n // wq
    shift_q = (wq - 1).bit_length()
    rpg = tile // GROUPS
    grp_refs = scratch[:GROUPS]
    scr2_ref = scratch[GROUPS]
    dummy = rpg * nq

    for g in range(GROUPS):
        grp_refs[g][...] = jnp.zeros_like(grp_refs[g])

    base = i * GROUPS
    starts = [bounds_ref[base + g] for g in range(GROUPS)]
    ends = [bounds_ref[base + g + 1] for g in range(GROUPS)]
    trip = functools.reduce(jnp.maximum,
                            [ends[g] - starts[g] for g in range(GROUPS)])

    iota = lax.broadcasted_iota(jnp.int32, (1, wq), 1)

    def body(j, carry):
        for jj in range(UNROLL_J):
            vals, tgts, ohs = [], [], []
            for g in range(GROUPS):
                idx = starts[g] + j * UNROLL_J + jj
                pred = idx < ends[g]
                l = edges_ref[jnp.minimum(idx, e_pad - 1)]
                rq = (lax.shift_right_logical(l, shift_q)
                      - (i * tile + g * rpg) * nq)
                cq = lax.bitwise_and(l, wq - 1)
                tgt = jnp.where(pred, rq, dummy)
                v = grp_refs[g][tgt]
                oh = jnp.where(iota == cq, 1.0, 0.0)
                vals.append(v); tgts.append(tgt); ohs.append(oh)
            for g in range(GROUPS):
                grp_refs[g][tgts[g]] = vals[g] + ohs[g]
        return carry

    lax.fori_loop(0, (trip + UNROLL_J - 1) // UNROLL_J, body, 0)

    for g in range(GROUPS):
        scr2_ref[g * rpg:(g + 1) * rpg, :] = (
            grp_refs[g][:rpg * nq].reshape(rpg, n))
    slab = scr2_ref[...]
    adj_ref[...] = slab.astype(jnp.int8)
    deg = jnp.sum(slab, axis=1, keepdims=True) + 1.0
    d = lax.rsqrt(deg)
    d_ref[...] = d

    xb = x_ref[...].astype(jnp.bfloat16)
    xw = jnp.dot(xb, w_ref[...], preferred_element_type=jnp.float32)
    m0_ref[...] = (xw * d).astype(m0_ref.dtype)


def _prop_kernel(adj_ref, d_ref, m_ref, o_ref, *, tile):
    i = pl.program_id(0)
    a = adj_ref[...].astype(jnp.bfloat16)
    h = jnp.dot(a, m_ref[...], preferred_element_type=jnp.float32)
    start = pl.multiple_of(i * tile, tile)
    h += m_ref[pl.ds(start, tile), :].astype(jnp.float32)
    d = d_ref[...]
    o_ref[...] = (h * (d * d)).astype(o_ref.dtype)


def _prop_final_kernel(adj_ref, d_ref, m_ref, b_ref, o_ref, *, tile,
                       num_classes):
    i = pl.program_id(0)
    a = adj_ref[...].astype(jnp.bfloat16)
    h = jnp.dot(a, m_ref[...], preferred_element_type=jnp.float32)
    start = pl.multiple_of(i * tile, tile)
    h += m_ref[pl.ds(start, tile), :].astype(jnp.float32)
    logits = h * d_ref[...] + b_ref[...]
    o_ref[...] = _masked_log_softmax(logits, num_classes).astype(o_ref.dtype)


def kernel(x, edge_index, weight, bias):
    n, f = x.shape
    c = weight.shape[1]
    row, col = edge_index[0], edge_index[1]
    e = int(row.shape[0])
    t = min(TILE, n)
    cp = ((c + LANE - 1) // LANE) * LANE
    e_pad = max(1 << (e + 1 - 1).bit_length(), 1024)

    lin = row * n + col
    lin = jnp.concatenate(
        [lin, jnp.full((e_pad - e,), n * n, jnp.int32)])
    sl = lax.sort(lin, is_stable=False)
    rows_per_group = t // GROUPS
    n_groups = (n // t) * GROUPS
    boundaries = jnp.arange(n_groups + 1, dtype=jnp.int32) * (rows_per_group * n)
    bounds = jnp.sum(
        (lin[:, None] < boundaries[None, :]).astype(jnp.int32), axis=0,
        dtype=jnp.int32)

    w_p = jnp.zeros((f, cp), jnp.bfloat16).at[:, :c].set(
        weight.astype(jnp.bfloat16))
    b_p = jnp.zeros((1, cp), jnp.float32).at[0, :c].set(bias)

    grid_rows = n // t
    vmem_limit = 100 * 1024 * 1024

    adj, d_p, m0 = pl.pallas_call(
        functools.partial(_build_kernel, n=n, tile=t, e_pad=e_pad),
        out_shape=[jax.ShapeDtypeStruct((n, n), jnp.int8),
                   jax.ShapeDtypeStruct((n, 1), jnp.float32),
                   jax.ShapeDtypeStruct((n, cp), jnp.bfloat16)],
        grid_spec=pltpu.PrefetchScalarGridSpec(
            num_scalar_prefetch=2,
            grid=(grid_rows,),
            in_specs=[pl.BlockSpec((t, f), lambda i, *_: (i, 0)),
                      pl.BlockSpec((f, cp), lambda i, *_: (0, 0))],
            out_specs=[pl.BlockSpec((t, n), lambda i, *_: (i, 0)),
                       pl.BlockSpec((t, 1), lambda i, *_: (i, 0)),
                       pl.BlockSpec((t, cp), lambda i, *_: (i, 0))],
            scratch_shapes=(
                [pltpu.VMEM(
                    ((t // GROUPS) * (n // min(8 * LANE, n)) + 8, 1,
                     min(8 * LANE, n)), jnp.float32)
                 for _ in range(GROUPS)]
                + [pltpu.VMEM((t, n), jnp.float32)]),
        ),
        compiler_params=pltpu.CompilerParams(
            dimension_semantics=("parallel",),
            vmem_limit_bytes=vmem_limit),
        cost_estimate=pl.CostEstimate(
            flops=2 * n * f * cp + 8 * e, transcendentals=n,
            bytes_accessed=n * n + n * f * 4 + f * cp * 2 + n * cp * 2),
    )(bounds, sl, x, w_p)

    m1 = pl.pallas_call(
        functools.partial(_prop_kernel, tile=t),
        out_shape=jax.ShapeDtypeStruct((n, cp), jnp.bfloat16),
        grid=(grid_rows,),
        in_specs=[pl.BlockSpec((t, n), lambda i: (i, 0)),
                  pl.BlockSpec((t, 1), lambda i: (i, 0)),
                  pl.BlockSpec((n, cp), lambda i: (0, 0))],
        out_specs=pl.BlockSpec((t, cp), lambda i: (i, 0)),
        compiler_params=pltpu.CompilerParams(
            dimension_semantics=("parallel",),
            vmem_limit_bytes=vmem_limit),
        cost_estimate=pl.CostEstimate(
            flops=2 * n * n * cp, transcendentals=0,
            bytes_accessed=n * n + n * cp * 2 + n * 4 + n * cp * 2),
    )(adj, d_p, m0)

    out_p = pl.pallas_call(
        functools.partial(_prop_final_kernel, tile=t, num_classes=c),
        out_shape=jax.ShapeDtypeStruct((n, cp), jnp.float32),
        grid=(grid_rows,),
        in_specs=[pl.BlockSpec((t, n), lambda i: (i, 0)),
                  pl.BlockSpec((t, 1), lambda i: (i, 0)),
                  pl.BlockSpec((n, cp), lambda i: (0, 0)),
                  pl.BlockSpec((1, cp), lambda i: (0, 0))],
        out_specs=pl.BlockSpec((t, cp), lambda i: (i, 0)),
        compiler_params=pltpu.CompilerParams(
            dimension_semantics=("parallel",),
            vmem_limit_bytes=vmem_limit),
        cost_estimate=pl.CostEstimate(
            flops=2 * n * n * cp, transcendentals=n * cp,
            bytes_accessed=n * n + n * cp * 2 + n * 4 + cp * 4 + n * cp * 4),
    )(adj, d_p, m1, b_p)

    return out_p[:, :c]
